# Initial kernel scaffold; baseline (speedup 1.0000x reference)
import jax
import jax.numpy as jnp
from jax.experimental import pallas as pl


def kernel(input, slices):
    raise NotImplementedError("write your pallas kernel here")


# speedup vs baseline: 3.8644x; 3.8644x over previous
import functools

import jax
import jax.numpy as jnp
from jax import lax
from jax.experimental import pallas as pl
from jax.experimental.pallas import tpu as pltpu
from jax.experimental.pallas import tpu_sc as plsc

_NUM_ITEMS = 8
_L = 512
_C = 128
_CHUNK = 32
_NCHUNK = _C // _CHUNK


def _build():
    info = plsc.get_sparse_core_info()
    nc = info.num_cores
    mesh = plsc.VectorSubcoreMesh(core_axis_name="c", subcore_axis_name="s")

    @functools.partial(
        pl.kernel,
        out_type=jax.ShapeDtypeStruct((_NUM_ITEMS, _C), jnp.float32),
        mesh=mesh,
        scratch_types=[
            pltpu.VMEM((16,), jnp.int32),
            pltpu.VMEM((_L, _CHUNK), jnp.float32),
            pltpu.VMEM((_CHUNK,), jnp.float32),
        ],
        compiler_params=pltpu.CompilerParams(
            use_tc_tiling_on_sc=False, needs_layout_passes=False),
    )
    def run(in_hbm, starts_hbm, out_hbm, starts_v, block_v, acc_v):
        wid = lax.axis_index("s") * nc + lax.axis_index("c")
        item = wid // _NCHUNK
        c0 = (wid % _NCHUNK) * _CHUNK

        pltpu.sync_copy(starts_hbm, starts_v)
        svec = starts_v[...]
        lanes = lax.iota(jnp.int32, 16)
        row = jnp.sum(jnp.where(lanes == 2 * item, svec, 0))

        pltpu.sync_copy(in_hbm.at[row, :, pl.ds(c0, _CHUNK)], block_v)

        zeros = jnp.zeros((16,), jnp.float32)

        def body(r, carry):
            a0, a1 = carry
            return (a0 + block_v[r, pl.ds(0, 16)],
                    a1 + block_v[r, pl.ds(16, 16)])

        a0, a1 = lax.fori_loop(0, _L, body, (zeros, zeros))
        acc_v[pl.ds(0, 16)] = a0
        acc_v[pl.ds(16, 16)] = a1
        pltpu.sync_copy(acc_v, out_hbm.at[item, pl.ds(c0, _CHUNK)])

    return run


_run = _build()


def kernel(input, slices):
    starts = slices.reshape(16).astype(jnp.int32)
    out = _run(input, starts)
    return tuple(out[i].reshape(1, _C, 1) for i in range(_NUM_ITEMS))


# speedup vs baseline: 3.9974x; 1.0344x over previous
import functools

import jax
import jax.numpy as jnp
from jax import lax
from jax.experimental import pallas as pl
from jax.experimental.pallas import tpu as pltpu
from jax.experimental.pallas import tpu_sc as plsc

_NUM_ITEMS = 8
_L = 512
_C = 128
_CHUNK = 32
_NCHUNK = _C // _CHUNK
_HALF = _L // 2
_UNROLL = 8


def _build():
    info = plsc.get_sparse_core_info()
    nc = info.num_cores
    mesh = plsc.VectorSubcoreMesh(core_axis_name="c", subcore_axis_name="s")

    @functools.partial(
        pl.kernel,
        out_type=jax.ShapeDtypeStruct((_NUM_ITEMS, _C), jnp.float32),
        mesh=mesh,
        scratch_types=[
            pltpu.VMEM((16,), jnp.int32),
            pltpu.VMEM((_L, _CHUNK), jnp.float32),
            pltpu.VMEM((_CHUNK,), jnp.float32),
        ],
        compiler_params=pltpu.CompilerParams(
            use_tc_tiling_on_sc=False, needs_layout_passes=False),
    )
    def run(in_hbm, starts_hbm, out_hbm, starts_v, block_v, acc_v):

        wid = lax.axis_index("s") * nc + lax.axis_index("c")
        item = wid // _NCHUNK
        c0 = (wid % _NCHUNK) * _CHUNK

        pltpu.sync_copy(starts_hbm, starts_v)
        svec = starts_v[...]
        lanes = lax.iota(jnp.int32, 16)
        row = jnp.sum(jnp.where(lanes == 2 * item, svec, 0))

        pltpu.sync_copy(in_hbm.at[row, :, pl.ds(c0, _CHUNK)], block_v)

        zeros = jnp.zeros((16,), jnp.float32)

        def body(t, carry):
            a00, a01, a10, a11 = carry
            r = t * _UNROLL
            for k in range(_UNROLL):
                x0 = block_v[r + k, pl.ds(0, 16)]
                x1 = block_v[r + k, pl.ds(16, 16)]
                if k % 2 == 0:
                    a00 = a00 + x0
                    a01 = a01 + x1
                else:
                    a10 = a10 + x0
                    a11 = a11 + x1
            return a00, a01, a10, a11

        a00, a01, a10, a11 = lax.fori_loop(
            0, _L // _UNROLL, body, (zeros, zeros, zeros, zeros))
        acc_v[pl.ds(0, 16)] = a00 + a10
        acc_v[pl.ds(16, 16)] = a01 + a11

        pltpu.sync_copy(acc_v, out_hbm.at[item, pl.ds(c0, _CHUNK)])

    return run


_run = _build()


def kernel(input, slices):
    starts = slices.reshape(16).astype(jnp.int32)
    out = _run(input, starts)
    return tuple(out[i].reshape(1, _C, 1) for i in range(_NUM_ITEMS))


# speedup vs baseline: 4.5017x; 1.1262x over previous
import functools

import jax
import jax.numpy as jnp
from jax import lax
from jax.experimental import pallas as pl
from jax.experimental.pallas import tpu as pltpu
from jax.experimental.pallas import tpu_sc as plsc

_NUM_ITEMS = 8
_L = 512
_C = 128
_CHUNK = 32
_NCHUNK = _C // _CHUNK
_HALF = _L // 2
_UNROLL = 8


def _build():
    info = plsc.get_sparse_core_info()
    nc = info.num_cores
    mesh = plsc.VectorSubcoreMesh(core_axis_name="c", subcore_axis_name="s")

    @functools.partial(
        pl.kernel,
        out_type=tuple(
            jax.ShapeDtypeStruct((1, _C), jnp.float32)
            for _ in range(_NUM_ITEMS)
        ),
        mesh=mesh,
        scratch_types=[
            pltpu.VMEM((16,), jnp.int32),
            pltpu.VMEM((_L, _CHUNK), jnp.float32),
            pltpu.VMEM((_CHUNK,), jnp.float32),
        ],
        compiler_params=pltpu.CompilerParams(
            use_tc_tiling_on_sc=False, needs_layout_passes=False),
    )
    def run(in_hbm, starts_hbm, *refs):
        outs = refs[:_NUM_ITEMS]
        starts_v, block_v, acc_v = refs[_NUM_ITEMS:]

        wid = lax.axis_index("s") * nc + lax.axis_index("c")
        item = wid // _NCHUNK
        c0 = (wid % _NCHUNK) * _CHUNK

        pltpu.sync_copy(starts_hbm, starts_v)
        svec = starts_v[...]
        lanes = lax.iota(jnp.int32, 16)
        row = jnp.sum(jnp.where(lanes == 2 * item, svec, 0))

        pltpu.sync_copy(in_hbm.at[row, :, pl.ds(c0, _CHUNK)], block_v)

        zeros = jnp.zeros((16,), jnp.float32)

        def body(t, carry):
            a00, a01, a10, a11 = carry
            r = t * _UNROLL
            for k in range(_UNROLL):
                x0 = block_v[r + k, pl.ds(0, 16)]
                x1 = block_v[r + k, pl.ds(16, 16)]
                if k % 2 == 0:
                    a00 = a00 + x0
                    a01 = a01 + x1
                else:
                    a10 = a10 + x0
                    a11 = a11 + x1
            return a00, a01, a10, a11

        a00, a01, a10, a11 = lax.fori_loop(
            0, _L // _UNROLL, body, (zeros, zeros, zeros, zeros))
        acc_v[pl.ds(0, 16)] = a00 + a10
        acc_v[pl.ds(16, 16)] = a01 + a11

        for k in range(_NUM_ITEMS):
            @pl.when(item == k)
            def _(k=k):
                pltpu.sync_copy(acc_v, outs[k].at[0, pl.ds(c0, _CHUNK)])

    return run


_run = _build()


def kernel(input, slices):
    starts = slices.reshape(16).astype(jnp.int32)
    return tuple(o.reshape(1, _C, 1) for o in _run(input, starts))
